# speedup vs baseline: 2.6739x; 2.6739x over previous
import jax
import jax.numpy as jnp
from jax.experimental import pallas as pl
from jax.experimental.pallas import tpu as pltpu

_NBL = 64
_NANT = 65
_NT = 128
_NF = 4096
_CF = 256


def _body(vm_ref, j_ref, out_ref):
    out_ref[...] = j_ref[0:_NBL] * vm_ref[...] * j_ref[1:_NANT]


def kernel(V_m, jones):
    vm3 = V_m.reshape(_NBL, _NT, _NF)
    j3 = jones.reshape(_NANT, _NT, _NF)
    grid = (_NF // _CF,)
    out = pl.pallas_call(
        _body,
        grid=grid,
        in_specs=[
            pl.BlockSpec((_NBL, _NT, _CF), lambda i: (0, 0, i)),
            pl.BlockSpec((_NANT, _NT, _CF), lambda i: (0, 0, i)),
        ],
        out_specs=pl.BlockSpec((_NBL, _NT, _CF), lambda i: (0, 0, i)),
        out_shape=jax.ShapeDtypeStruct((_NBL, _NT, _NF), jnp.float32),
    )(vm3, j3)
    return out.reshape(1, 1, _NBL, _NT, _NF)
